# baseline (speedup 1.0000x reference)
import functools

import jax
import jax.numpy as jnp
from jax import lax
from jax.experimental import pallas as pl
from jax.experimental.pallas import tpu as pltpu
from jax.experimental.pallas import tpu_sc as plsc

N = 10000
NP = 10240
D = 128
DH = 512
E = 320000
K = 128
NTILES = 32
CPT = 80
IB = 16
NBLK = CPT // IB
EP = NTILES * CPT * K
ROWS_PT = NP // 16

K2 = 32
CPT2 = 320
IB2 = 16
NBLK2 = CPT2 // IB2
NSLOT = 8

_mesh = plsc.VectorSubcoreMesh(core_axis_name="c", subcore_axis_name="s")


@functools.partial(
    pl.kernel,
    out_type=jax.ShapeDtypeStruct((2, NP), jnp.float32),
    mesh=_mesh,
    scratch_types=[
        pltpu.VMEM((CPT, K), jnp.int32),
        pltpu.VMEM((K,), jnp.float32),
        pltpu.VMEM((ROWS_PT,), jnp.float32),
        pltpu.VMEM_SHARED((NP,), jnp.float32),
    ],
)
def _deg_kernel(dst_hbm, deg_hbm, idx_v, ones_v, zero_v, sh):
    cid = lax.axis_index("c")
    sid = lax.axis_index("s")
    wid = cid * 16 + sid
    for i in range(K // 16):
        ones_v[pl.ds(i * 16, 16)] = jnp.ones((16,), jnp.float32)

    def _zero(i, carry):
        zero_v[pl.ds(i * 16, 16)] = jnp.zeros((16,), jnp.float32)
        return carry

    lax.fori_loop(0, ROWS_PT // 16, _zero, 0)
    pltpu.sync_copy(zero_v, sh.at[pl.ds(sid * ROWS_PT, ROWS_PT)])
    pltpu.sync_copy(dst_hbm.at[pl.ds(wid * CPT, CPT)], idx_v)
    plsc.subcore_barrier()

    def _scat(j, carry):
        pltpu.sync_copy(ones_v, sh.at[idx_v.at[j]], add=True)
        return carry

    lax.fori_loop(0, CPT, _scat, 0)
    plsc.subcore_barrier()
    pltpu.sync_copy(
        sh.at[pl.ds(sid * ROWS_PT, ROWS_PT)],
        deg_hbm.at[cid, pl.ds(sid * ROWS_PT, ROWS_PT)],
    )


@functools.partial(
    pl.kernel,
    out_type=jax.ShapeDtypeStruct((2, NP, D), jnp.float32),
    mesh=_mesh,
    scratch_types=[
        pltpu.VMEM((2, IB2, K2), jnp.int32),
        pltpu.VMEM((2, IB2, K2), jnp.int32),
        pltpu.VMEM((NSLOT, K2, D), jnp.float32),
        pltpu.VMEM_SHARED((NP, D), jnp.float32),
        pltpu.SemaphoreType.DMA,
        pltpu.SemaphoreType.DMA,
        pltpu.SemaphoreType.DMA,
        pltpu.SemaphoreType.DMA,
        pltpu.SemaphoreType.DMA,
        pltpu.SemaphoreType.DMA,
        pltpu.SemaphoreType.DMA,
        pltpu.SemaphoreType.DMA,
        pltpu.SemaphoreType.DMA,
    ],
)
def _agg_kernel(xs_hbm, src_hbm, dst_hbm, agg_hbm, sidx_v, didx_v, rows_v, sh,
                sga, sgb, sgc, sgd, sge, sgf, sgg, sgh, sidm):
    cid = lax.axis_index("c")
    sid = lax.axis_index("s")
    wid = cid * 16 + sid
    sems = (sga, sgb, sgc, sgd, sge, sgf, sgg, sgh)

    def _zero(i, carry):
        rows_v[0, i // 8, pl.ds((i % 8) * 16, 16)] = jnp.zeros((16,), jnp.float32)
        return carry

    lax.fori_loop(0, K2 * D // 16, _zero, 0)
    for t in range(ROWS_PT // K2):
        pltpu.sync_copy(rows_v.at[0], sh.at[pl.ds(sid * ROWS_PT + t * K2, K2)])
    plsc.subcore_barrier()

    base = wid * CPT2
    pltpu.sync_copy(src_hbm.at[pl.ds(base, IB2)], sidx_v.at[0])
    pltpu.sync_copy(dst_hbm.at[pl.ds(base, IB2)], didx_v.at[0])
    pltpu.async_copy(src_hbm.at[pl.ds(base + IB2, IB2)], sidx_v.at[1], sidm)
    pltpu.async_copy(dst_hbm.at[pl.ds(base + IB2, IB2)], didx_v.at[1], sidm)
    for t in range(NSLOT):
        pltpu.async_copy(xs_hbm.at[sidx_v.at[0, t]], rows_v.at[t], sems[t])

    for bi in range(NBLK2):
        s = bi % 2

        def _quad(p, c2):
            j0 = NSLOT * p
            for t in range(NSLOT):
                pltpu.make_async_copy(xs_hbm.at[pl.ds(0, K2)], rows_v.at[t],
                                      sems[t]).wait()
                pltpu.sync_copy(rows_v.at[t], sh.at[didx_v.at[s, j0 + t]],
                                add=True)
                pltpu.async_copy(xs_hbm.at[sidx_v.at[s, j0 + NSLOT + t]],
                                 rows_v.at[t], sems[t])
            return c2

        lax.fori_loop(0, IB2 // NSLOT - 1, _quad, 0)

        if bi + 1 < NBLK2:
            ns = (bi + 1) % 2
            pltpu.make_async_copy(src_hbm.at[pl.ds(0, IB2)], sidx_v.at[ns],
                                  sidm).wait()
            pltpu.make_async_copy(dst_hbm.at[pl.ds(0, IB2)], didx_v.at[ns],
                                  sidm).wait()
            for t in range(NSLOT):
                pltpu.make_async_copy(xs_hbm.at[pl.ds(0, K2)], rows_v.at[t],
                                      sems[t]).wait()
                pltpu.sync_copy(rows_v.at[t],
                                sh.at[didx_v.at[s, IB2 - NSLOT + t]],
                                add=True)
                pltpu.async_copy(xs_hbm.at[sidx_v.at[ns, t]], rows_v.at[t],
                                 sems[t])
            if bi + 2 < NBLK2:
                nb = base + (bi + 2) * IB2
                pltpu.async_copy(src_hbm.at[pl.ds(nb, IB2)], sidx_v.at[s],
                                 sidm)
                pltpu.async_copy(dst_hbm.at[pl.ds(nb, IB2)], didx_v.at[s],
                                 sidm)
        else:
            for t in range(NSLOT):
                pltpu.make_async_copy(xs_hbm.at[pl.ds(0, K2)], rows_v.at[t],
                                      sems[t]).wait()
                pltpu.sync_copy(rows_v.at[t],
                                sh.at[didx_v.at[s, IB2 - NSLOT + t]],
                                add=True)
    plsc.subcore_barrier()
    pltpu.sync_copy(
        sh.at[pl.ds(sid * ROWS_PT, ROWS_PT)],
        agg_hbm.at[cid, pl.ds(sid * ROWS_PT, ROWS_PT)],
    )


def _scale_body(deg_ref, x_ref, dinv_ref, xs_ref):
    deg = deg_ref[0, :] + deg_ref[1, :] + 1.0
    dinv = lax.rsqrt(deg)
    dinv_ref[...] = dinv[:, None]
    xs_ref[...] = x_ref[...] * dinv[:, None]


_SCALE_ROWS = 1280


def _scale(degp, xp):
    return pl.pallas_call(
        _scale_body,
        grid=(NP // _SCALE_ROWS,),
        in_specs=[
            pl.BlockSpec((2, _SCALE_ROWS), lambda i: (0, i)),
            pl.BlockSpec((_SCALE_ROWS, D), lambda i: (i, 0)),
        ],
        out_specs=[
            pl.BlockSpec((_SCALE_ROWS, 1), lambda i: (i, 0)),
            pl.BlockSpec((_SCALE_ROWS, D), lambda i: (i, 0)),
        ],
        out_shape=[
            jax.ShapeDtypeStruct((NP, 1), jnp.float32),
            jax.ShapeDtypeStruct((NP, D), jnp.float32),
        ],
    )(degp, xp)


def _mm_body(agg_ref, xs_ref, dinv_ref, w_ref, b_ref, pw_ref, out_ref):
    pre = (agg_ref[0] + agg_ref[1] + xs_ref[...]) * dinv_ref[...]
    acc = jnp.dot(pre, w_ref[...], preferred_element_type=jnp.float32)
    acc = acc + b_ref[...][None, :]
    pw = pw_ref[0, 0]
    out_ref[...] = jnp.where(acc >= 0.0, acc, pw * acc)


_MM_ROWS = 400


def _matmul(agg, xs, dinv, W, b, pw):
    return pl.pallas_call(
        _mm_body,
        grid=(N // _MM_ROWS,),
        in_specs=[
            pl.BlockSpec((2, _MM_ROWS, D), lambda i: (0, i, 0)),
            pl.BlockSpec((_MM_ROWS, D), lambda i: (i, 0)),
            pl.BlockSpec((_MM_ROWS, 1), lambda i: (i, 0)),
            pl.BlockSpec((D, DH), lambda i: (0, 0)),
            pl.BlockSpec((DH,), lambda i: (0,)),
            pl.BlockSpec((1, 1), lambda i: (0, 0)),
        ],
        out_specs=pl.BlockSpec((_MM_ROWS, DH), lambda i: (i, 0)),
        out_shape=jax.ShapeDtypeStruct((N, DH), jnp.float32),
    )(agg, xs, dinv, W, b, pw)


def kernel(x, edge_index, W, b, prelu_weight):
    src = edge_index[0]
    dst = edge_index[1]
    pad = N + jnp.arange(EP - E, dtype=jnp.int32) % (NP - N)
    srcf = jnp.concatenate([src, pad])
    dstf = jnp.concatenate([dst, pad])
    dstp = dstf.reshape(NTILES * CPT, K)
    xp = jnp.concatenate([x, jnp.zeros((NP - N, D), dtype=x.dtype)], axis=0)

    degp = _deg_kernel(dstp)
    dinv, xs = _scale(degp, xp)
    agg = _agg_kernel(xs, srcf.reshape(NTILES * CPT2, K2),
                      dstf.reshape(NTILES * CPT2, K2))
    return _matmul(agg, xs, dinv, W, b, prelu_weight.reshape(1, 1))


# speedup vs baseline: 1.0140x; 1.0140x over previous
import functools

import jax
import jax.numpy as jnp
from jax import lax
from jax.experimental import pallas as pl
from jax.experimental.pallas import tpu as pltpu
from jax.experimental.pallas import tpu_sc as plsc

N = 10000
NP = 10240
D = 128
DH = 512
E = 320000
K = 128
NTILES = 32
CPT = 80
IB = 16
NBLK = CPT // IB
EP = NTILES * CPT * K
ROWS_PT = NP // 16

K2 = 64
CPT2 = 160
IB2 = 32
NBLK2 = CPT2 // IB2
NSLOT = 4

_mesh = plsc.VectorSubcoreMesh(core_axis_name="c", subcore_axis_name="s")


@functools.partial(
    pl.kernel,
    out_type=jax.ShapeDtypeStruct((2, NP), jnp.float32),
    mesh=_mesh,
    scratch_types=[
        pltpu.VMEM((CPT, K), jnp.int32),
        pltpu.VMEM((K,), jnp.float32),
        pltpu.VMEM((ROWS_PT,), jnp.float32),
        pltpu.VMEM_SHARED((NP,), jnp.float32),
    ],
)
def _deg_kernel(dst_hbm, deg_hbm, idx_v, ones_v, zero_v, sh):
    cid = lax.axis_index("c")
    sid = lax.axis_index("s")
    wid = cid * 16 + sid
    for i in range(K // 16):
        ones_v[pl.ds(i * 16, 16)] = jnp.ones((16,), jnp.float32)

    def _zero(i, carry):
        zero_v[pl.ds(i * 16, 16)] = jnp.zeros((16,), jnp.float32)
        return carry

    lax.fori_loop(0, ROWS_PT // 16, _zero, 0)
    pltpu.sync_copy(zero_v, sh.at[pl.ds(sid * ROWS_PT, ROWS_PT)])
    pltpu.sync_copy(dst_hbm.at[pl.ds(wid * CPT, CPT)], idx_v)
    plsc.subcore_barrier()

    def _scat(j, carry):
        pltpu.sync_copy(ones_v, sh.at[idx_v.at[j]], add=True)
        return carry

    lax.fori_loop(0, CPT, _scat, 0)
    plsc.subcore_barrier()
    pltpu.sync_copy(
        sh.at[pl.ds(sid * ROWS_PT, ROWS_PT)],
        deg_hbm.at[cid, pl.ds(sid * ROWS_PT, ROWS_PT)],
    )


@functools.partial(
    pl.kernel,
    out_type=jax.ShapeDtypeStruct((2, NP, D), jnp.float32),
    mesh=_mesh,
    scratch_types=[
        pltpu.VMEM((2, IB2, K2), jnp.int32),
        pltpu.VMEM((2, IB2, K2), jnp.int32),
        pltpu.VMEM((NSLOT, K2, D), jnp.float32),
        pltpu.VMEM_SHARED((NP, D), jnp.float32),
        pltpu.SemaphoreType.DMA,
        pltpu.SemaphoreType.DMA,
        pltpu.SemaphoreType.DMA,
        pltpu.SemaphoreType.DMA,
        pltpu.SemaphoreType.DMA,
    ],
)
def _agg_kernel(xs_hbm, src_hbm, dst_hbm, agg_hbm, sidx_v, didx_v, rows_v, sh,
                sga, sgb, sgc, sgd, sidm):
    cid = lax.axis_index("c")
    sid = lax.axis_index("s")
    wid = cid * 16 + sid
    sems = (sga, sgb, sgc, sgd)

    def _zero(i, carry):
        rows_v[0, i // 8, pl.ds((i % 8) * 16, 16)] = jnp.zeros((16,), jnp.float32)
        return carry

    lax.fori_loop(0, K2 * D // 16, _zero, 0)
    for t in range(ROWS_PT // K2):
        pltpu.sync_copy(rows_v.at[0], sh.at[pl.ds(sid * ROWS_PT + t * K2, K2)])
    plsc.subcore_barrier()

    base = wid * CPT2
    pltpu.sync_copy(src_hbm.at[pl.ds(base, IB2)], sidx_v.at[0])
    pltpu.sync_copy(dst_hbm.at[pl.ds(base, IB2)], didx_v.at[0])
    pltpu.async_copy(src_hbm.at[pl.ds(base + IB2, IB2)], sidx_v.at[1], sidm)
    pltpu.async_copy(dst_hbm.at[pl.ds(base + IB2, IB2)], didx_v.at[1], sidm)
    for t in range(NSLOT):
        pltpu.async_copy(xs_hbm.at[sidx_v.at[0, t]], rows_v.at[t], sems[t])

    for bi in range(NBLK2):
        s = bi % 2

        def _quad(p, c2):
            j0 = NSLOT * p
            for t in range(NSLOT):
                pltpu.make_async_copy(xs_hbm.at[pl.ds(0, K2)], rows_v.at[t],
                                      sems[t]).wait()
                pltpu.sync_copy(rows_v.at[t], sh.at[didx_v.at[s, j0 + t]],
                                add=True)
                pltpu.async_copy(xs_hbm.at[sidx_v.at[s, j0 + NSLOT + t]],
                                 rows_v.at[t], sems[t])
            return c2

        lax.fori_loop(0, IB2 // NSLOT - 1, _quad, 0)

        if bi + 1 < NBLK2:
            ns = (bi + 1) % 2
            pltpu.make_async_copy(src_hbm.at[pl.ds(0, IB2)], sidx_v.at[ns],
                                  sidm).wait()
            pltpu.make_async_copy(dst_hbm.at[pl.ds(0, IB2)], didx_v.at[ns],
                                  sidm).wait()
            for t in range(NSLOT):
                pltpu.make_async_copy(xs_hbm.at[pl.ds(0, K2)], rows_v.at[t],
                                      sems[t]).wait()
                pltpu.sync_copy(rows_v.at[t],
                                sh.at[didx_v.at[s, IB2 - NSLOT + t]],
                                add=True)
                pltpu.async_copy(xs_hbm.at[sidx_v.at[ns, t]], rows_v.at[t],
                                 sems[t])
            if bi + 2 < NBLK2:
                nb = base + (bi + 2) * IB2
                pltpu.async_copy(src_hbm.at[pl.ds(nb, IB2)], sidx_v.at[s],
                                 sidm)
                pltpu.async_copy(dst_hbm.at[pl.ds(nb, IB2)], didx_v.at[s],
                                 sidm)
        else:
            for t in range(NSLOT):
                pltpu.make_async_copy(xs_hbm.at[pl.ds(0, K2)], rows_v.at[t],
                                      sems[t]).wait()
                pltpu.sync_copy(rows_v.at[t],
                                sh.at[didx_v.at[s, IB2 - NSLOT + t]],
                                add=True)
    plsc.subcore_barrier()
    pltpu.sync_copy(
        sh.at[pl.ds(sid * ROWS_PT, ROWS_PT)],
        agg_hbm.at[cid, pl.ds(sid * ROWS_PT, ROWS_PT)],
    )


def _scale_body(deg_ref, x_ref, dinv_ref, xs_ref):
    deg = deg_ref[0, :] + deg_ref[1, :] + 1.0
    dinv = lax.rsqrt(deg)
    dinv_ref[...] = dinv[:, None]
    xs_ref[...] = x_ref[...] * dinv[:, None]


_SCALE_ROWS = 1280


def _scale(degp, xp):
    return pl.pallas_call(
        _scale_body,
        grid=(NP // _SCALE_ROWS,),
        in_specs=[
            pl.BlockSpec((2, _SCALE_ROWS), lambda i: (0, i)),
            pl.BlockSpec((_SCALE_ROWS, D), lambda i: (i, 0)),
        ],
        out_specs=[
            pl.BlockSpec((_SCALE_ROWS, 1), lambda i: (i, 0)),
            pl.BlockSpec((_SCALE_ROWS, D), lambda i: (i, 0)),
        ],
        out_shape=[
            jax.ShapeDtypeStruct((NP, 1), jnp.float32),
            jax.ShapeDtypeStruct((NP, D), jnp.float32),
        ],
    )(degp, xp)


def _mm_body(agg_ref, xs_ref, dinv_ref, w_ref, b_ref, pw_ref, out_ref):
    pre = (agg_ref[0] + agg_ref[1] + xs_ref[...]) * dinv_ref[...]
    acc = jnp.dot(pre, w_ref[...], preferred_element_type=jnp.float32)
    acc = acc + b_ref[...][None, :]
    pw = pw_ref[0, 0]
    out_ref[...] = jnp.where(acc >= 0.0, acc, pw * acc)


_MM_ROWS = 400


def _matmul(agg, xs, dinv, W, b, pw):
    return pl.pallas_call(
        _mm_body,
        grid=(N // _MM_ROWS,),
        in_specs=[
            pl.BlockSpec((2, _MM_ROWS, D), lambda i: (0, i, 0)),
            pl.BlockSpec((_MM_ROWS, D), lambda i: (i, 0)),
            pl.BlockSpec((_MM_ROWS, 1), lambda i: (i, 0)),
            pl.BlockSpec((D, DH), lambda i: (0, 0)),
            pl.BlockSpec((DH,), lambda i: (0,)),
            pl.BlockSpec((1, 1), lambda i: (0, 0)),
        ],
        out_specs=pl.BlockSpec((_MM_ROWS, DH), lambda i: (i, 0)),
        out_shape=jax.ShapeDtypeStruct((N, DH), jnp.float32),
    )(agg, xs, dinv, W, b, pw)


def kernel(x, edge_index, W, b, prelu_weight):
    src = edge_index[0]
    dst = edge_index[1]
    pad = N + jnp.arange(EP - E, dtype=jnp.int32) % (NP - N)
    srcf = jnp.concatenate([src, pad])
    dstf = jnp.concatenate([dst, pad])
    dstp = dstf.reshape(NTILES * CPT, K)
    xp = jnp.concatenate([x, jnp.zeros((NP - N, D), dtype=x.dtype)], axis=0)

    degp = _deg_kernel(dstp)
    dinv, xs = _scale(degp, xp)
    agg = _agg_kernel(xs, srcf.reshape(NTILES * CPT2, K2),
                      dstf.reshape(NTILES * CPT2, K2))
    return _matmul(agg, xs, dinv, W, b, prelu_weight.reshape(1, 1))
